# baseline (speedup 1.0000x reference)
import functools
import numpy as np
import jax
import jax.numpy as jnp
from jax.experimental import pallas as pl
from jax.experimental.pallas import tpu as pltpu

N_NODES = 10000
NUM_HEADS = 8
MAP_FEATS = 64


def _proj_kernel(x_ref, wq_ref, wk_ref, wv_ref, wm_ref, bm_ref,
                 q_ref, k_ref, v_ref, m_ref):
    x = x_ref[...]
    q_ref[...] = jnp.dot(x, wq_ref[...], preferred_element_type=jnp.float32)
    k_ref[...] = jnp.dot(x, wk_ref[...], preferred_element_type=jnp.float32)
    v_ref[...] = jnp.dot(x, wv_ref[...], preferred_element_type=jnp.float32)
    m_ref[...] = jnp.dot(x, wm_ref[...], preferred_element_type=jnp.float32) + bm_ref[...]


def _proj(x, p):
    N, D = x.shape
    Hq = p['Wq'].shape[1]
    Hv = p['Wv'].shape[1]
    Hm = p['Wm'].shape[1]
    BN = 1000
    grid = (N // BN,)
    return pl.pallas_call(
        _proj_kernel,
        grid=grid,
        in_specs=[
            pl.BlockSpec((BN, D), lambda i: (i, 0)),
            pl.BlockSpec((D, Hq), lambda i: (0, 0)),
            pl.BlockSpec((D, Hq), lambda i: (0, 0)),
            pl.BlockSpec((D, Hv), lambda i: (0, 0)),
            pl.BlockSpec((D, Hm), lambda i: (0, 0)),
            pl.BlockSpec((Hm,), lambda i: (0,)),
        ],
        out_specs=[
            pl.BlockSpec((BN, Hq), lambda i: (i, 0)),
            pl.BlockSpec((BN, Hq), lambda i: (i, 0)),
            pl.BlockSpec((BN, Hv), lambda i: (i, 0)),
            pl.BlockSpec((BN, Hm), lambda i: (i, 0)),
        ],
        out_shape=[
            jax.ShapeDtypeStruct((N, Hq), jnp.float32),
            jax.ShapeDtypeStruct((N, Hq), jnp.float32),
            jax.ShapeDtypeStruct((N, Hv), jnp.float32),
            jax.ShapeDtypeStruct((N, Hm), jnp.float32),
        ],
    )(x, p['Wq'], p['Wk'], p['Wv'], p['Wm'], p['bm'])


def _out_kernel(x_ref, mmax_ref, mmean_ref, h_ref, wg_ref, bg_ref,
                wo_ref, bo_ref, o_ref):
    x = x_ref[...]
    gin = jnp.concatenate([x, mmax_ref[...], mmean_ref[...]], axis=1)
    gate = jax.nn.sigmoid(jnp.dot(gin, wg_ref[...], preferred_element_type=jnp.float32) + bg_ref[...])
    N = x.shape[0]
    K = gate.shape[1]
    h = h_ref[...].reshape(N, K, -1)
    hg = (gate[:, :, None] * h).reshape(N, -1)
    oin = jnp.concatenate([x, hg], axis=1)
    o_ref[...] = jnp.dot(oin, wo_ref[...], preferred_element_type=jnp.float32) + bo_ref[...]


def _outstage(x, mmax, mmean, h, p):
    N, D = x.shape
    Hm = mmax.shape[1]
    K = p['Wg'].shape[1]
    O = p['Wo'].shape[1]
    Dh = h.shape[1]
    BN = 1000
    return pl.pallas_call(
        _out_kernel,
        grid=(N // BN,),
        in_specs=[
            pl.BlockSpec((BN, D), lambda i: (i, 0)),
            pl.BlockSpec((BN, Hm), lambda i: (i, 0)),
            pl.BlockSpec((BN, D), lambda i: (i, 0)),
            pl.BlockSpec((BN, Dh), lambda i: (i, 0)),
            pl.BlockSpec(p['Wg'].shape, lambda i: (0, 0)),
            pl.BlockSpec((K,), lambda i: (0,)),
            pl.BlockSpec(p['Wo'].shape, lambda i: (0, 0)),
            pl.BlockSpec((O,), lambda i: (0,)),
        ],
        out_specs=pl.BlockSpec((BN, O), lambda i: (i, 0)),
        out_shape=jax.ShapeDtypeStruct((N, O), jnp.float32),
    )(x, mmax, mmean, h, p['Wg'], p['bg'], p['Wo'], p['bo'])


def _layer(p, x, src, dst):
    N = x.shape[0]
    K, da = NUM_HEADS, MAP_FEATS
    q, k, v, m = _proj(x, p)
    dv = v.shape[1] // K
    q = q.reshape(N, K, da)
    k = k.reshape(N, K, da)
    v = v.reshape(N, K, dv)
    e = jnp.sum(q[dst] * k[src], axis=-1) / np.sqrt(da)
    emax = jax.ops.segment_max(e, dst, num_segments=N)
    emax = jnp.where(jnp.isfinite(emax), emax, 0.0)
    ee = jnp.exp(e - emax[dst])
    esum = jax.ops.segment_sum(ee, dst, num_segments=N)
    alpha = ee / (esum[dst] + 1e-9)
    msg = alpha[:, :, None] * v[src]
    h = jax.ops.segment_sum(msg.reshape(msg.shape[0], K * dv), dst, num_segments=N)
    mmax = jax.ops.segment_max(m[src], dst, num_segments=N)
    mmax = jnp.where(jnp.isfinite(mmax), mmax, 0.0)
    ssum = jax.ops.segment_sum(x[src], dst, num_segments=N)
    deg = jax.ops.segment_sum(jnp.ones((src.shape[0],), x.dtype), dst, num_segments=N)
    mmean = ssum / jnp.maximum(deg, 1.0)[:, None]
    return _outstage(x, mmax, mmean, h, p)


def kernel(macro_features, edge_index, params):
    src = edge_index[0]
    dst = edge_index[1]
    h = macro_features
    for i, p in enumerate(params):
        h = _layer(p, h, src, dst)
        if i < len(params) - 1:
            h = jnp.tanh(h)
    return jax.nn.relu(h)
